# baseline (speedup 1.0000x reference)
import functools

import jax
import jax.numpy as jnp
from jax import lax
from jax.experimental import pallas as pl
from jax.experimental.pallas import tpu as pltpu
from jax.experimental.pallas import tpu_sc as plsc

_B, _T, _N = 16, 12, 1024
_BT = _B * _T
_NW = 32
_NCHUNK = _BT * (_N // 128) // _NW
_STEPS = 288


def _sc_body(x_hbm, w_hbm, b_hbm, tod_hbm, dow_hbm, node_hbm, adp_hbm,
             out_hbm,
             xb0, ab0, xb1, ab1, xinb, tdb,
             nodeb, tod_v, dow_v, wv, bv,
             sr0, sw0, sr1, sw1, sv):
    c = lax.axis_index("c")
    s = lax.axis_index("s")
    wid = s * 2 + c
    n0 = lax.div(wid, 4) * 128
    g0 = wid * _NCHUNK

    pltpu.sync_copy(w_hbm, wv)
    pltpu.sync_copy(b_hbm, bv)
    pltpu.sync_copy(tod_hbm, tod_v)
    pltpu.sync_copy(dow_hbm, dow_v)
    pltpu.sync_copy(node_hbm.at[:, pl.ds(n0, 128)], nodeb)
    w0 = [wv[0, pl.ds(k * 16, 16)] for k in range(8)]
    w1 = [wv[1, pl.ds(k * 16, 16)] for k in range(8)]
    w2 = [wv[2, pl.ds(k * 16, 16)] for k in range(8)]
    bb = [bv[pl.ds(k * 16, 16)] for k in range(8)]

    sets = ((xb0, ab0, sr0, sw0), (xb1, ab1, sr1, sw1))

    def chunk_bt(g):
        return lax.rem(g0 + g, _BT)

    def valu_write_copies(g):
        bt = chunk_bt(g)
        nds = pl.ds(n0, 128)
        return (
            pltpu.make_async_copy(xinb, out_hbm.at[bt, pl.ds(0, 128), nds], sv),
            pltpu.make_async_copy(tdb.at[pl.ds(0, 64), :],
                                  out_hbm.at[bt, pl.ds(128, 64), nds], sv),
            pltpu.make_async_copy(tdb.at[pl.ds(64, 64), :],
                                  out_hbm.at[bt, pl.ds(192, 64), nds], sv),
        )

    def write_copies(st, g):
        (xb, ab, sr, sw) = st
        bt = chunk_bt(g)
        nds = pl.ds(n0, 128)
        return (
            pltpu.make_async_copy(nodeb,
                                  out_hbm.at[bt, pl.ds(256, 64), nds], sw),
            pltpu.make_async_copy(ab,
                                  out_hbm.at[bt, pl.ds(320, 128), nds], sw),
        )

    def fire_reads(st, g):
        (xb, ab, sr, sw) = st
        bt = chunk_bt(g)
        t = lax.rem(bt, _T)
        pltpu.make_async_copy(x_hbm.at[bt, :, pl.ds(n0, 128)], xb, sr).start()
        pltpu.make_async_copy(adp_hbm.at[pl.ds(t * 128, 128), pl.ds(n0, 128)],
                              ab, sr).start()

    def wait_reads(st):
        (xb, ab, sr, sw) = st
        pltpu.make_async_copy(x_hbm.at[0, :, pl.ds(0, 128)], xb, sr).wait()
        pltpu.make_async_copy(adp_hbm.at[pl.ds(0, 128), pl.ds(0, 128)], ab,
                              sr).wait()

    def process(st, g):
        (xb, ab, sr, sw) = st
        wait_reads(st)
        @pl.when(g >= 1)
        def _():
            for cp in valu_write_copies(g - 1):
                cp.wait()

        @plsc.parallel_loop(0, 8)
        def grp_body(ng):
            nds = pl.ds(ng * 16, 16)
            x0v = xb[0, nds]
            x1v = xb[1, nds]
            x2v = xb[2, nds]
            tiv = (x1v * float(_STEPS)).astype(jnp.int32)
            div = (x2v * 7.0).astype(jnp.int32)
            for k in range(8):
                for lane in range(16):
                    cc = k * 16 + lane
                    xinb[cc, nds] = (x0v * w0[k][lane] + x1v * w1[k][lane]
                                     + x2v * w2[k][lane] + bb[k][lane])
            for cc in range(64):
                tdb[cc, nds] = plsc.load_gather(tod_v, [tiv + cc * _STEPS])
            for cc in range(64):
                tdb[64 + cc, nds] = plsc.load_gather(dow_v, [div + cc * 7])
        for cp in valu_write_copies(g) + write_copies(st, g):
            cp.start()

    def step(k, rd_set, pr_set):
        @pl.when(k >= 2)
        def _():
            for cp in write_copies(rd_set, k - 2):
                cp.wait()

        @pl.when(k < _NCHUNK)
        def _():
            fire_reads(rd_set, k)

        @pl.when(k >= 1)
        def _():
            process(pr_set, k - 1)

    def body(k, carry):
        @pl.when(lax.rem(k, 2) == 0)
        def _():
            step(k, sets[0], sets[1])

        @pl.when(lax.rem(k, 2) == 1)
        def _():
            step(k, sets[1], sets[0])
        return 0

    lax.fori_loop(0, _NCHUNK + 1, body, 0)
    for cp in valu_write_copies(_NCHUNK - 1):
        cp.wait()
    for cp in write_copies(sets[(_NCHUNK - 1) % 2], _NCHUNK - 1):
        cp.wait()


def _set_types():
    return [
        pltpu.VMEM((3, 128), jnp.float32),
        pltpu.VMEM((128, 128), jnp.float32),
    ]


_sc_kernel = functools.partial(
    pl.kernel,
    mesh=plsc.VectorSubcoreMesh(core_axis_name="c", subcore_axis_name="s"),
    out_type=jax.ShapeDtypeStruct((_BT, 448, _N), jnp.float32),
    compiler_params=pltpu.CompilerParams(use_tc_tiling_on_sc=True,
                                         needs_layout_passes=False),
    scratch_types=_set_types() + _set_types() + [
        pltpu.VMEM((128, 128), jnp.float32),
        pltpu.VMEM((128, 128), jnp.float32),
        pltpu.VMEM((64, 128), jnp.float32),
        pltpu.VMEM((288 * 64,), jnp.float32),
        pltpu.VMEM((7 * 64,), jnp.float32),
        pltpu.VMEM((3, 128), jnp.float32),
        pltpu.VMEM((128,), jnp.float32),
        pltpu.SemaphoreType.DMA,
        pltpu.SemaphoreType.DMA,
        pltpu.SemaphoreType.DMA,
        pltpu.SemaphoreType.DMA,
        pltpu.SemaphoreType.DMA,
    ],
)(_sc_body)


@jax.jit
def kernel(x, W_in, b_in, tod_table, dow_table, node_emb, adaptive_emb):
    B, T, N, _ = x.shape
    x_t = x.transpose(0, 1, 3, 2).reshape(_BT, 3, N)
    adp_t = adaptive_emb.transpose(0, 2, 1).reshape(T * 128, N)
    out_t = _sc_kernel(x_t, W_in, b_in, tod_table.T.reshape(-1),
                       dow_table.T.reshape(-1), node_emb.T, adp_t)
    return out_t.reshape(B, T, 448, N).transpose(0, 1, 3, 2)


# speedup vs baseline: 1.4965x; 1.4965x over previous
import functools

import jax
import jax.numpy as jnp
from jax import lax
from jax.experimental import pallas as pl
from jax.experimental.pallas import tpu as pltpu
from jax.experimental.pallas import tpu_sc as plsc

_B, _T, _N = 16, 12, 1024
_BT = _B * _T
_NW = 32
_NCHUNK = _BT * (_N // 128) // _NW
_STEPS = 288


def _sc_body(x_hbm, w_hbm, b_hbm, tod_hbm, dow_hbm, node_hbm, adp_hbm,
             out_hbm,
             xb0, ab0, xb1, ab1, xinb, tdb,
             nodeb, tod_v, dow_v, wv, bv,
             sr0, sw0, sr1, sw1, sv):
    c = lax.axis_index("c")
    s = lax.axis_index("s")
    wid = s * 2 + c
    n0 = lax.div(wid, 4) * 128
    g0 = wid * _NCHUNK

    pltpu.sync_copy(w_hbm, wv)
    pltpu.sync_copy(b_hbm, bv)
    pltpu.sync_copy(tod_hbm, tod_v)
    pltpu.sync_copy(dow_hbm, dow_v)
    pltpu.sync_copy(node_hbm.at[:, pl.ds(n0, 128)], nodeb)
    w0 = [wv[0, pl.ds(k * 16, 16)] for k in range(8)]
    w1 = [wv[1, pl.ds(k * 16, 16)] for k in range(8)]
    w2 = [wv[2, pl.ds(k * 16, 16)] for k in range(8)]
    bb = [bv[pl.ds(k * 16, 16)] for k in range(8)]

    sets = ((xb0, ab0, sr0, sw0), (xb1, ab1, sr1, sw1))

    def chunk_bt(g):
        return lax.rem(g0 + g, _BT)

    def valu_write_copies(g):
        bt = chunk_bt(g)
        nds = pl.ds(n0, 128)
        return (
            pltpu.make_async_copy(xinb, out_hbm.at[bt, pl.ds(0, 128), nds], sv),
            pltpu.make_async_copy(tdb.at[pl.ds(0, 64), :],
                                  out_hbm.at[bt, pl.ds(128, 64), nds], sv),
            pltpu.make_async_copy(tdb.at[pl.ds(64, 64), :],
                                  out_hbm.at[bt, pl.ds(192, 64), nds], sv),
        )

    def write_copies(st, g):
        (xb, ab, sr, sw) = st
        bt = chunk_bt(g)
        nds = pl.ds(n0, 128)
        return (
            pltpu.make_async_copy(nodeb,
                                  out_hbm.at[bt, pl.ds(256, 64), nds], sw),
            pltpu.make_async_copy(ab,
                                  out_hbm.at[bt, pl.ds(320, 128), nds], sw),
        )

    def fire_reads(st, g):
        (xb, ab, sr, sw) = st
        bt = chunk_bt(g)
        t = lax.rem(bt, _T)
        pltpu.make_async_copy(x_hbm.at[bt, :, pl.ds(n0, 128)], xb, sr).start()
        pltpu.make_async_copy(adp_hbm.at[pl.ds(t * 128, 128), pl.ds(n0, 128)],
                              ab, sr).start()

    def wait_reads(st):
        (xb, ab, sr, sw) = st
        pltpu.make_async_copy(x_hbm.at[0, :, pl.ds(0, 128)], xb, sr).wait()
        pltpu.make_async_copy(adp_hbm.at[pl.ds(0, 128), pl.ds(0, 128)], ab,
                              sr).wait()

    def process(st, g):
        (xb, ab, sr, sw) = st
        wait_reads(st)
        @pl.when(g >= 1)
        def _():
            for cp in valu_write_copies(g - 1):
                cp.wait()

        xv0 = [xb[0, pl.ds(ng * 16, 16)] for ng in range(8)]
        xv1 = [xb[1, pl.ds(ng * 16, 16)] for ng in range(8)]
        xv2 = [xb[2, pl.ds(ng * 16, 16)] for ng in range(8)]
        tivs = [(v * float(_STEPS)).astype(jnp.int32) for v in xv1]
        divs = [(v * 7.0).astype(jnp.int32) for v in xv2]

        def xin_body(kk, _):
            w0v = wv[0, pl.ds(kk * 16, 16)]
            w1v = wv[1, pl.ds(kk * 16, 16)]
            w2v = wv[2, pl.ds(kk * 16, 16)]
            bbv = bv[pl.ds(kk * 16, 16)]
            for lane in range(16):
                cc = kk * 16 + lane
                w0s = w0v[lane]
                w1s = w1v[lane]
                w2s = w2v[lane]
                bs = bbv[lane]
                for ng in range(8):
                    xinb[cc, pl.ds(ng * 16, 16)] = (
                        xv0[ng] * w0s + xv1[ng] * w1s + xv2[ng] * w2s + bs)
            return 0

        lax.fori_loop(0, 8, xin_body, 0)

        def tab_body(kk, _):
            for lane in range(16):
                cc = kk * 16 + lane
                for ng in range(8):
                    tdb[cc, pl.ds(ng * 16, 16)] = plsc.load_gather(
                        tod_v, [tivs[ng] + cc * _STEPS])
                for ng in range(8):
                    tdb[64 + cc, pl.ds(ng * 16, 16)] = plsc.load_gather(
                        dow_v, [divs[ng] + cc * 7])
            return 0

        lax.fori_loop(0, 4, tab_body, 0)
        for cp in valu_write_copies(g) + write_copies(st, g):
            cp.start()

    def step(k, rd_set, pr_set):
        @pl.when(k >= 2)
        def _():
            for cp in write_copies(rd_set, k - 2):
                cp.wait()

        @pl.when(k < _NCHUNK)
        def _():
            fire_reads(rd_set, k)

        @pl.when(k >= 1)
        def _():
            process(pr_set, k - 1)

    def body(k, carry):
        @pl.when(lax.rem(k, 2) == 0)
        def _():
            step(k, sets[0], sets[1])

        @pl.when(lax.rem(k, 2) == 1)
        def _():
            step(k, sets[1], sets[0])
        return 0

    lax.fori_loop(0, _NCHUNK + 1, body, 0)
    for cp in valu_write_copies(_NCHUNK - 1):
        cp.wait()
    for cp in write_copies(sets[(_NCHUNK - 1) % 2], _NCHUNK - 1):
        cp.wait()


def _set_types():
    return [
        pltpu.VMEM((3, 128), jnp.float32),
        pltpu.VMEM((128, 128), jnp.float32),
    ]


_sc_kernel = functools.partial(
    pl.kernel,
    mesh=plsc.VectorSubcoreMesh(core_axis_name="c", subcore_axis_name="s"),
    out_type=jax.ShapeDtypeStruct((_BT, 448, _N), jnp.float32),
    compiler_params=pltpu.CompilerParams(use_tc_tiling_on_sc=True,
                                         needs_layout_passes=False),
    scratch_types=_set_types() + _set_types() + [
        pltpu.VMEM((128, 128), jnp.float32),
        pltpu.VMEM((128, 128), jnp.float32),
        pltpu.VMEM((64, 128), jnp.float32),
        pltpu.VMEM((288 * 64,), jnp.float32),
        pltpu.VMEM((7 * 64,), jnp.float32),
        pltpu.VMEM((3, 128), jnp.float32),
        pltpu.VMEM((128,), jnp.float32),
        pltpu.SemaphoreType.DMA,
        pltpu.SemaphoreType.DMA,
        pltpu.SemaphoreType.DMA,
        pltpu.SemaphoreType.DMA,
        pltpu.SemaphoreType.DMA,
    ],
)(_sc_body)


@jax.jit
def kernel(x, W_in, b_in, tod_table, dow_table, node_emb, adaptive_emb):
    B, T, N, _ = x.shape
    x_t = x.transpose(0, 1, 3, 2).reshape(_BT, 3, N)
    adp_t = adaptive_emb.transpose(0, 2, 1).reshape(T * 128, N)
    out_t = _sc_kernel(x_t, W_in, b_in, tod_table.T.reshape(-1),
                       dow_table.T.reshape(-1), node_emb.T, adp_t)
    return out_t.reshape(B, T, 448, N).transpose(0, 1, 3, 2)
